# baseline (speedup 1.0000x reference)
import functools

import jax
import jax.numpy as jnp
from jax import lax
from jax.experimental import pallas as pl
from jax.experimental.pallas import tpu as pltpu
from jax.experimental.pallas import tpu_sc as plsc

_NC = 2
_NS = 16
_NW = _NC * _NS
_C = 512
_G = 128


def _sc_embedding_gather(idx_flat, table):
    B = idx_flat.shape[0]
    D = table.shape[1]
    b_per_w = B // _NW
    n_chunks = b_per_w // _C

    mesh = plsc.VectorSubcoreMesh(core_axis_name="c", subcore_axis_name="s")

    @functools.partial(
        pl.kernel,
        mesh=mesh,
        out_type=jax.ShapeDtypeStruct((B, D), jnp.float32),
        scratch_types=[
            pltpu.VMEM((_C,), jnp.int32),
            pltpu.VMEM((_C, D), jnp.float32),
            pltpu.SemaphoreType.DMA,
        ],
        compiler_params=pltpu.CompilerParams(use_tc_tiling_on_sc=False),
    )
    def k(idx_hbm, table_hbm, out_hbm, idx_v, rows_v, sem):
        wid = lax.axis_index("s") * _NC + lax.axis_index("c")
        base = wid * b_per_w

        def step(i, carry):
            off = base + i * _C
            pltpu.sync_copy(idx_hbm.at[pl.ds(off, _C)], idx_v)
            copies = [
                pltpu.async_copy(
                    table_hbm.at[idx_v.at[pl.ds(j * _G, _G)]],
                    rows_v.at[pl.ds(j * _G, _G)],
                    sem,
                )
                for j in range(_C // _G)
            ]
            for c in copies:
                c.wait()
            pltpu.sync_copy(rows_v, out_hbm.at[pl.ds(off, _C)])
            return carry

        lax.fori_loop(0, n_chunks, step, 0)

    return k(idx_flat, table)


def kernel(word_idxs, we_weight):
    b, s = word_idxs.shape
    d = we_weight.shape[1]
    idx_flat = word_idxs.reshape(-1)
    out = _sc_embedding_gather(idx_flat, we_weight)
    return out.reshape(b, s, d)


# speedup vs baseline: 1.0406x; 1.0406x over previous
import functools

import jax
import jax.numpy as jnp
from jax import lax
from jax.experimental import pallas as pl
from jax.experimental.pallas import tpu as pltpu
from jax.experimental.pallas import tpu_sc as plsc

_NC = 2
_NS = 16
_NW = _NC * _NS
_C = 256
_G = 128
_NBUF = 4


def _sc_embedding_gather(idx_flat, table):
    B = idx_flat.shape[0]
    D = table.shape[1]
    b_per_w = B // _NW
    n_chunks = b_per_w // _C

    mesh = plsc.VectorSubcoreMesh(core_axis_name="c", subcore_axis_name="s")

    @functools.partial(
        pl.kernel,
        mesh=mesh,
        out_type=jax.ShapeDtypeStruct((B, D), jnp.float32),
        scratch_types=[
            pltpu.VMEM((b_per_w,), jnp.int32),
            pltpu.VMEM((_NBUF, _C, D), jnp.float32),
            pltpu.SemaphoreType.DMA((_NBUF,)),
            pltpu.SemaphoreType.DMA((_NBUF,)),
        ],
        compiler_params=pltpu.CompilerParams(use_tc_tiling_on_sc=False),
    )
    def k(idx_hbm, table_hbm, out_hbm, idx_v, rows_v, gsem, wsem):
        wid = lax.axis_index("s") * _NC + lax.axis_index("c")
        base = wid * b_per_w
        pltpu.sync_copy(idx_hbm.at[pl.ds(base, b_per_w)], idx_v)

        def fire_gather(i, b):
            for j in range(_C // _G):
                pltpu.async_copy(
                    table_hbm.at[idx_v.at[pl.ds(i * _C + j * _G, _G)]],
                    rows_v.at[b].at[pl.ds(j * _G, _G)],
                    gsem.at[b],
                )

        for b in range(_NBUF):
            fire_gather(b, b)

        def step(i, carry):
            b = lax.rem(i, _NBUF)
            pltpu.make_async_copy(
                out_hbm.at[pl.ds(0, _C)], rows_v.at[b], gsem.at[b]
            ).wait()
            pltpu.async_copy(
                rows_v.at[b], out_hbm.at[pl.ds(base + i * _C, _C)], wsem.at[b]
            )
            j = i - 1

            @pl.when(jnp.logical_and(j >= 0, j + _NBUF < n_chunks))
            def _():
                bj = lax.rem(j + _NBUF, _NBUF)
                pltpu.make_async_copy(
                    out_hbm.at[pl.ds(0, _C)], rows_v.at[bj], wsem.at[bj]
                ).wait()
                fire_gather(j + _NBUF, bj)

            return carry

        lax.fori_loop(0, n_chunks, step, 0)

        for b in range(_NBUF):
            pltpu.make_async_copy(
                out_hbm.at[pl.ds(0, _C)], rows_v.at[b], wsem.at[b]
            ).wait()

    return k(idx_flat, table)


def kernel(word_idxs, we_weight):
    b, s = word_idxs.shape
    d = we_weight.shape[1]
    idx_flat = word_idxs.reshape(-1)
    out = _sc_embedding_gather(idx_flat, we_weight)
    return out.reshape(b, s, d)


# speedup vs baseline: 1.0417x; 1.0011x over previous
import functools

import jax
import jax.numpy as jnp
from jax import lax
from jax.experimental import pallas as pl
from jax.experimental.pallas import tpu as pltpu
from jax.experimental.pallas import tpu_sc as plsc

_NC = 2
_NS = 16
_NW = _NC * _NS
_NBUF = 4
_G = 128


def _sc_embedding_gather(word_idxs, table):
    R, S = word_idxs.shape
    D = table.shape[1]
    r_per_w = R // _NW

    mesh = plsc.VectorSubcoreMesh(core_axis_name="c", subcore_axis_name="s")

    @functools.partial(
        pl.kernel,
        mesh=mesh,
        out_type=jax.ShapeDtypeStruct((R, S, D), jnp.float32),
        scratch_types=[
            pltpu.VMEM((r_per_w, S), jnp.int32),
            pltpu.VMEM((_NBUF, S, D), jnp.float32),
            pltpu.SemaphoreType.DMA((_NBUF,)),
            pltpu.SemaphoreType.DMA((_NBUF,)),
        ],
        compiler_params=pltpu.CompilerParams(use_tc_tiling_on_sc=False),
    )
    def k(idx_hbm, table_hbm, out_hbm, idx_v, rows_v, gsem, wsem):
        wid = lax.axis_index("s") * _NC + lax.axis_index("c")
        base = wid * r_per_w
        pltpu.sync_copy(idx_hbm.at[pl.ds(base, r_per_w), :], idx_v)

        def fire_gather(i, b):
            for j0 in range(0, S, _G):
                g = min(_G, S - j0)
                pltpu.async_copy(
                    table_hbm.at[idx_v.at[i, pl.ds(j0, g)]],
                    rows_v.at[b].at[pl.ds(j0, g)],
                    gsem.at[b],
                )

        for b in range(_NBUF):
            fire_gather(b, b)

        def step(i, carry):
            b = lax.rem(i, _NBUF)
            pltpu.make_async_copy(
                out_hbm.at[0], rows_v.at[b], gsem.at[b]
            ).wait()
            pltpu.async_copy(rows_v.at[b], out_hbm.at[base + i], wsem.at[b])
            j = i - 1

            @pl.when(jnp.logical_and(j >= 0, j + _NBUF < r_per_w))
            def _():
                bj = lax.rem(j + _NBUF, _NBUF)
                pltpu.make_async_copy(
                    out_hbm.at[0], rows_v.at[bj], wsem.at[bj]
                ).wait()
                fire_gather(j + _NBUF, bj)

            return carry

        lax.fori_loop(0, r_per_w, step, 0)

        for b in range(_NBUF):
            pltpu.make_async_copy(
                out_hbm.at[0], rows_v.at[b], wsem.at[b]
            ).wait()

    return k(word_idxs, table)


def kernel(word_idxs, we_weight):
    return _sc_embedding_gather(word_idxs, we_weight)


# speedup vs baseline: 1.3811x; 1.3258x over previous
import functools

import jax
import jax.numpy as jnp
from jax import lax
from jax.experimental import pallas as pl
from jax.experimental.pallas import tpu as pltpu
from jax.experimental.pallas import tpu_sc as plsc

_NC = 2
_NS = 16
_NW = _NC * _NS
_NBUF = 4
_G = 128


def _sc_embedding_gather(word_idxs, table):
    R, S = word_idxs.shape
    D = table.shape[1]
    r_per_w = R // _NW

    mesh = plsc.VectorSubcoreMesh(core_axis_name="c", subcore_axis_name="s")

    @functools.partial(
        pl.kernel,
        mesh=mesh,
        out_type=jax.ShapeDtypeStruct((R, S, 128), jnp.float32),
        scratch_types=[
            pltpu.VMEM((r_per_w, S), jnp.int32),
            pltpu.VMEM((_NBUF, S, D), jnp.float32),
            pltpu.SemaphoreType.DMA((_NBUF,)),
            pltpu.SemaphoreType.DMA((_NBUF,)),
        ],
        compiler_params=pltpu.CompilerParams(use_tc_tiling_on_sc=False),
    )
    def k(idx_hbm, table_hbm, out_hbm, idx_v, rows_v, gsem, wsem):
        wid = lax.axis_index("s") * _NC + lax.axis_index("c")
        base = wid * r_per_w
        pltpu.sync_copy(idx_hbm.at[pl.ds(base, r_per_w), :], idx_v)

        def fire_gather(i, b):
            for j0 in range(0, S, _G):
                g = min(_G, S - j0)
                pltpu.async_copy(
                    table_hbm.at[idx_v.at[i, pl.ds(j0, g)]],
                    rows_v.at[b].at[pl.ds(j0, g)],
                    gsem.at[b],
                )

        for b in range(_NBUF):
            fire_gather(b, b)

        def step(i, carry):
            b = lax.rem(i, _NBUF)
            pltpu.make_async_copy(
                out_hbm.at[0, :, pl.ds(0, D)], rows_v.at[b], gsem.at[b]
            ).wait()
            pltpu.async_copy(
                rows_v.at[b], out_hbm.at[base + i, :, pl.ds(0, D)], wsem.at[b]
            )
            j = i - 1

            @pl.when(jnp.logical_and(j >= 0, j + _NBUF < r_per_w))
            def _():
                bj = lax.rem(j + _NBUF, _NBUF)
                pltpu.make_async_copy(
                    out_hbm.at[0, :, pl.ds(0, D)], rows_v.at[bj], wsem.at[bj]
                ).wait()
                fire_gather(j + _NBUF, bj)

            return carry

        lax.fori_loop(0, r_per_w, step, 0)

        for b in range(_NBUF):
            pltpu.make_async_copy(
                out_hbm.at[0, :, pl.ds(0, D)], rows_v.at[b], wsem.at[b]
            ).wait()

    return k(word_idxs, table)[:, :, :64]


def kernel(word_idxs, we_weight):
    return _sc_embedding_gather(word_idxs, we_weight)


# speedup vs baseline: 1.5579x; 1.1281x over previous
import functools

import jax
import jax.numpy as jnp
from jax import lax
from jax.experimental import pallas as pl
from jax.experimental.pallas import tpu as pltpu
from jax.experimental.pallas import tpu_sc as plsc

_NC = 2
_NS = 16
_NW = _NC * _NS
_NBUF = 4
_G = 128
_DP = 128
_TC_BLK = 4096


def _tc_transpose_pad(wt):
    d, v = wt.shape
    grid = -(-v // _TC_BLK)

    def body(in_ref, out_ref):
        t = in_ref[...].T
        z = jnp.zeros((_TC_BLK, _DP - d), dtype=t.dtype)
        out_ref[...] = jnp.concatenate([t, z], axis=1).reshape(-1)

    return pl.pallas_call(
        body,
        grid=(grid,),
        in_specs=[pl.BlockSpec((d, _TC_BLK), lambda g: (0, g))],
        out_specs=pl.BlockSpec((_TC_BLK * _DP,), lambda g: (g,)),
        out_shape=jax.ShapeDtypeStruct((v * _DP,), jnp.float32),
    )(wt)


def _sc_embedding_gather(word_idxs, table128):
    R, S = word_idxs.shape
    r_per_w = R // _NW

    mesh = plsc.VectorSubcoreMesh(core_axis_name="c", subcore_axis_name="s")

    @functools.partial(
        pl.kernel,
        mesh=mesh,
        out_type=jax.ShapeDtypeStruct((R, S, _DP), jnp.float32),
        scratch_types=[
            pltpu.VMEM((r_per_w, S), jnp.int32),
            pltpu.VMEM((_NBUF, S, _DP), jnp.float32),
            pltpu.SemaphoreType.DMA((_NBUF,)),
            pltpu.SemaphoreType.DMA((_NBUF,)),
        ],
        compiler_params=pltpu.CompilerParams(use_tc_tiling_on_sc=False),
    )
    def k(idx_hbm, table_hbm, out_hbm, idx_v, rows_v, gsem, wsem):
        wid = lax.axis_index("s") * _NC + lax.axis_index("c")
        base = wid * r_per_w
        pltpu.sync_copy(idx_hbm.at[pl.ds(base, r_per_w), :], idx_v)

        def fire_gather(i, b):
            for j0 in range(0, S, _G):
                g = min(_G, S - j0)
                pltpu.async_copy(
                    table_hbm.at[idx_v.at[i, pl.ds(j0, g)]],
                    rows_v.at[b].at[pl.ds(j0, g)],
                    gsem.at[b],
                )

        for b in range(_NBUF):
            fire_gather(b, b)

        def step(i, carry):
            b = lax.rem(i, _NBUF)
            pltpu.make_async_copy(
                out_hbm.at[0], rows_v.at[b], gsem.at[b]
            ).wait()
            pltpu.async_copy(rows_v.at[b], out_hbm.at[base + i], wsem.at[b])
            j = i - 1

            @pl.when(jnp.logical_and(j >= 0, j + _NBUF < r_per_w))
            def _():
                bj = lax.rem(j + _NBUF, _NBUF)
                pltpu.make_async_copy(
                    out_hbm.at[0], rows_v.at[bj], wsem.at[bj]
                ).wait()
                fire_gather(j + _NBUF, bj)

            return carry

        lax.fori_loop(0, r_per_w, step, 0)

        for b in range(_NBUF):
            pltpu.make_async_copy(
                out_hbm.at[0], rows_v.at[b], wsem.at[b]
            ).wait()

    return k(word_idxs, table128)


def kernel(word_idxs, we_weight):
    v, d = we_weight.shape
    table128 = _tc_transpose_pad(we_weight.T).reshape(v, _DP)
    out = _sc_embedding_gather(word_idxs, table128)
    return out[:, :, :d]


# speedup vs baseline: 1.8833x; 1.2089x over previous
import functools

import jax
import jax.numpy as jnp
from jax import lax
from jax.experimental import pallas as pl
from jax.experimental.pallas import tpu as pltpu
from jax.experimental.pallas import tpu_sc as plsc

_NC = 2
_NS = 16
_NW = _NC * _NS
_NBUF = 4
_G = 128
_DP = 128
_TC_BLK = 4096


def _tc_transpose_pad(wt):
    d, v = wt.shape
    grid = -(-v // _TC_BLK)

    def body(in_ref, out_ref):
        t = in_ref[...].T
        z = jnp.zeros((_TC_BLK, _DP - d), dtype=t.dtype)
        out_ref[...] = jnp.concatenate([t, z], axis=1).reshape(-1)

    return pl.pallas_call(
        body,
        grid=(grid,),
        in_specs=[pl.BlockSpec((d, _TC_BLK), lambda g: (0, g))],
        out_specs=pl.BlockSpec((_TC_BLK * _DP,), lambda g: (g,)),
        out_shape=jax.ShapeDtypeStruct((v * _DP,), jnp.float32),
    )(wt)


def _sc_embedding_gather(word_idxs, table):
    R, S = word_idxs.shape
    D = table.shape[1]
    r_per_w = R // _NW

    mesh = plsc.VectorSubcoreMesh(core_axis_name="c", subcore_axis_name="s")

    @functools.partial(
        pl.kernel,
        mesh=mesh,
        out_type=jax.ShapeDtypeStruct((R, S, _DP), jnp.float32),
        scratch_types=[
            pltpu.VMEM((r_per_w, S), jnp.int32),
            pltpu.VMEM((_NBUF, S, 64), jnp.float32),
            pltpu.SemaphoreType.DMA((_NBUF,)),
            pltpu.SemaphoreType.DMA((_NBUF,)),
        ],
        compiler_params=pltpu.CompilerParams(use_tc_tiling_on_sc=False),
    )
    def k(idx_hbm, table_hbm, out_hbm, idx_v, rows_v, gsem, wsem):
        wid = lax.axis_index("s") * _NC + lax.axis_index("c")
        base = wid * r_per_w
        pltpu.sync_copy(idx_hbm.at[pl.ds(base, r_per_w), :], idx_v)

        def fire_gather(i, b):
            for j0 in range(0, S, _G):
                g = min(_G, S - j0)
                pltpu.async_copy(
                    table_hbm.at[idx_v.at[i, pl.ds(j0, g)]],
                    rows_v.at[b].at[pl.ds(j0, g)],
                    gsem.at[b],
                )

        for b in range(_NBUF):
            fire_gather(b, b)

        def step(i, carry):
            b = lax.rem(i, _NBUF)
            pltpu.make_async_copy(
                out_hbm.at[0, :, pl.ds(0, D)], rows_v.at[b], gsem.at[b]
            ).wait()
            pltpu.async_copy(
                rows_v.at[b], out_hbm.at[base + i, :, pl.ds(0, D)], wsem.at[b]
            )
            j = i - 1

            @pl.when(jnp.logical_and(j >= 0, j + _NBUF < r_per_w))
            def _():
                bj = lax.rem(j + _NBUF, _NBUF)
                pltpu.make_async_copy(
                    out_hbm.at[0, :, pl.ds(0, D)], rows_v.at[bj], wsem.at[bj]
                ).wait()
                fire_gather(j + _NBUF, bj)

            return carry

        lax.fori_loop(0, r_per_w, step, 0)

        for b in range(_NBUF):
            pltpu.make_async_copy(
                out_hbm.at[0, :, pl.ds(0, D)], rows_v.at[b], wsem.at[b]
            ).wait()

    return k(word_idxs, table)


def kernel(word_idxs, we_weight):
    v, d = we_weight.shape
    table2 = _tc_transpose_pad(we_weight.T).reshape(2 * v, d)
    out = _sc_embedding_gather(word_idxs * 2, table2)
    return out[:, :, :d]


# speedup vs baseline: 2.1142x; 1.1226x over previous
import functools

import jax
import jax.numpy as jnp
from jax import lax
from jax.experimental import pallas as pl
from jax.experimental.pallas import tpu as pltpu
from jax.experimental.pallas import tpu_sc as plsc

_NC = 2
_NS = 16
_NW = _NC * _NS
_NBUF = 4
_G = 128
_DP = 128
_TC_BLK = 8192


def _tc_transpose_pad(wt):
    d, v = wt.shape
    grid = -(-v // _TC_BLK)

    def body(in_ref, out_ref):
        t = in_ref[...].T
        z = jnp.zeros((_TC_BLK, _DP - d), dtype=t.dtype)
        out_ref[...] = jnp.concatenate([t, z], axis=1).reshape(-1)

    return pl.pallas_call(
        body,
        grid=(grid,),
        in_specs=[pl.BlockSpec((d, _TC_BLK), lambda g: (0, g))],
        out_specs=pl.BlockSpec((_TC_BLK * _DP,), lambda g: (g,)),
        out_shape=jax.ShapeDtypeStruct((v * _DP,), jnp.float32),
    )(wt)


def _sc_embedding_gather(word_idxs, table):
    R, S = word_idxs.shape
    D = table.shape[1]
    r_per_w = R // _NW

    mesh = plsc.VectorSubcoreMesh(core_axis_name="c", subcore_axis_name="s")

    @functools.partial(
        pl.kernel,
        mesh=mesh,
        out_type=jax.ShapeDtypeStruct((R, S, _DP), jnp.float32),
        scratch_types=[
            pltpu.VMEM((r_per_w, S), jnp.int32),
            pltpu.VMEM((_NBUF, S, 64), jnp.float32),
            pltpu.SemaphoreType.DMA((_NBUF,)),
            pltpu.SemaphoreType.DMA((_NBUF,)),
        ],
        compiler_params=pltpu.CompilerParams(use_tc_tiling_on_sc=False),
    )
    def k(idx_hbm, table_hbm, out_hbm, idx_v, rows_v, gsem, wsem):
        wid = lax.axis_index("s") * _NC + lax.axis_index("c")
        base = wid * r_per_w
        pltpu.sync_copy(idx_hbm.at[pl.ds(base, r_per_w), :], idx_v)

        def fire_gather(i, b):
            for j0 in range(0, S, _G):
                g = min(_G, S - j0)
                pltpu.async_copy(
                    table_hbm.at[idx_v.at[i, pl.ds(j0, g)]],
                    rows_v.at[b].at[pl.ds(j0, g)],
                    gsem.at[b],
                )

        for b in range(_NBUF):
            fire_gather(b, b)

        def step(i, carry):
            b = lax.rem(i, _NBUF)
            pltpu.make_async_copy(
                out_hbm.at[0, :, pl.ds(0, D)], rows_v.at[b], gsem.at[b]
            ).wait()
            pltpu.async_copy(
                rows_v.at[b], out_hbm.at[base + i, :, pl.ds(0, D)], wsem.at[b]
            )
            j = i - 1

            @pl.when(jnp.logical_and(j >= 0, j + _NBUF < r_per_w))
            def _():
                bj = lax.rem(j + _NBUF, _NBUF)
                pltpu.make_async_copy(
                    out_hbm.at[0, :, pl.ds(0, D)], rows_v.at[bj], wsem.at[bj]
                ).wait()
                fire_gather(j + _NBUF, bj)

            return carry

        lax.fori_loop(0, r_per_w, step, 0)

        for b in range(_NBUF):
            pltpu.make_async_copy(
                out_hbm.at[0, :, pl.ds(0, D)], rows_v.at[b], wsem.at[b]
            ).wait()

    return k(word_idxs, table)


def kernel(word_idxs, we_weight):
    v, d = we_weight.shape
    table2 = _tc_transpose_pad(we_weight.T).reshape(2 * v, d)
    out = _sc_embedding_gather(word_idxs * 2, table2)
    return out[:, :, :d]


# speedup vs baseline: 2.1830x; 1.0326x over previous
import functools

import jax
import jax.numpy as jnp
from jax import lax
from jax.experimental import pallas as pl
from jax.experimental.pallas import tpu as pltpu
from jax.experimental.pallas import tpu_sc as plsc

_NC = 2
_NS = 16
_NW = _NC * _NS
_NBUF = 4
_G = 128
_DP = 128
_TC_BLK = 16384


def _tc_transpose_pad(wt):
    d, v = wt.shape
    grid = -(-v // _TC_BLK)

    def body(in_ref, out_ref):
        t = in_ref[...].T
        z = jnp.zeros((_TC_BLK, _DP - d), dtype=t.dtype)
        out_ref[...] = jnp.concatenate([t, z], axis=1).reshape(-1)

    return pl.pallas_call(
        body,
        grid=(grid,),
        in_specs=[pl.BlockSpec((d, _TC_BLK), lambda g: (0, g))],
        out_specs=pl.BlockSpec((_TC_BLK * _DP,), lambda g: (g,)),
        out_shape=jax.ShapeDtypeStruct((v * _DP,), jnp.float32),
    )(wt)


def _sc_embedding_gather(word_idxs, table):
    R, S = word_idxs.shape
    D = table.shape[1]
    r_per_w = R // _NW

    mesh = plsc.VectorSubcoreMesh(core_axis_name="c", subcore_axis_name="s")

    @functools.partial(
        pl.kernel,
        mesh=mesh,
        out_type=jax.ShapeDtypeStruct((R, S, _DP), jnp.float32),
        scratch_types=[
            pltpu.VMEM((r_per_w, S), jnp.int32),
            pltpu.VMEM((_NBUF, S, 64), jnp.float32),
            pltpu.SemaphoreType.DMA((_NBUF,)),
            pltpu.SemaphoreType.DMA((_NBUF,)),
        ],
        compiler_params=pltpu.CompilerParams(use_tc_tiling_on_sc=False),
    )
    def k(idx_hbm, table_hbm, out_hbm, idx_v, rows_v, gsem, wsem):
        wid = lax.axis_index("s") * _NC + lax.axis_index("c")
        base = wid * r_per_w
        pltpu.sync_copy(idx_hbm.at[pl.ds(base, r_per_w), :], idx_v)

        def fire_gather(i, b):
            for j0 in range(0, S, _G):
                g = min(_G, S - j0)
                pltpu.async_copy(
                    table_hbm.at[idx_v.at[i, pl.ds(j0, g)]],
                    rows_v.at[b].at[pl.ds(j0, g)],
                    gsem.at[b],
                )

        for b in range(_NBUF):
            fire_gather(b, b)

        def step(i, carry):
            b = lax.rem(i, _NBUF)
            pltpu.make_async_copy(
                out_hbm.at[0, :, pl.ds(0, D)], rows_v.at[b], gsem.at[b]
            ).wait()
            pltpu.async_copy(
                rows_v.at[b], out_hbm.at[base + i, :, pl.ds(0, D)], wsem.at[b]
            )
            j = i - 1

            @pl.when(jnp.logical_and(j >= 0, j + _NBUF < r_per_w))
            def _():
                bj = lax.rem(j + _NBUF, _NBUF)
                pltpu.make_async_copy(
                    out_hbm.at[0, :, pl.ds(0, D)], rows_v.at[bj], wsem.at[bj]
                ).wait()
                fire_gather(j + _NBUF, bj)

            return carry

        lax.fori_loop(0, r_per_w, step, 0)

        for b in range(_NBUF):
            pltpu.make_async_copy(
                out_hbm.at[0, :, pl.ds(0, D)], rows_v.at[b], wsem.at[b]
            ).wait()

    return k(word_idxs, table)


def kernel(word_idxs, we_weight):
    v, d = we_weight.shape
    table2 = _tc_transpose_pad(we_weight.T).reshape(2 * v, d)
    out = _sc_embedding_gather(word_idxs * 2, table2)
    return out[:, :, :d]


# speedup vs baseline: 2.2070x; 1.0110x over previous
import functools

import jax
import jax.numpy as jnp
from jax import lax
from jax.experimental import pallas as pl
from jax.experimental.pallas import tpu as pltpu
from jax.experimental.pallas import tpu_sc as plsc

_NC = 2
_NS = 16
_NW = _NC * _NS
_NBUF = 4
_G = 128
_DP = 128
_TC_BLK = 32768


def _tc_transpose_pad(wt):
    d, v = wt.shape
    grid = -(-v // _TC_BLK)

    def body(in_ref, out_ref):
        t = in_ref[...].T
        z = jnp.zeros((_TC_BLK, _DP - d), dtype=t.dtype)
        out_ref[...] = jnp.concatenate([t, z], axis=1).reshape(-1)

    return pl.pallas_call(
        body,
        grid=(grid,),
        in_specs=[pl.BlockSpec((d, _TC_BLK), lambda g: (0, g))],
        out_specs=pl.BlockSpec((_TC_BLK * _DP,), lambda g: (g,)),
        out_shape=jax.ShapeDtypeStruct((v * _DP,), jnp.float32),
    )(wt)


def _sc_embedding_gather(word_idxs, table):
    R, S = word_idxs.shape
    D = table.shape[1]
    r_per_w = R // _NW

    mesh = plsc.VectorSubcoreMesh(core_axis_name="c", subcore_axis_name="s")

    @functools.partial(
        pl.kernel,
        mesh=mesh,
        out_type=jax.ShapeDtypeStruct((R, S, _DP), jnp.float32),
        scratch_types=[
            pltpu.VMEM((r_per_w, S), jnp.int32),
            pltpu.VMEM((_NBUF, S, 64), jnp.float32),
            pltpu.SemaphoreType.DMA((_NBUF,)),
            pltpu.SemaphoreType.DMA((_NBUF,)),
        ],
        compiler_params=pltpu.CompilerParams(use_tc_tiling_on_sc=False),
    )
    def k(idx_hbm, table_hbm, out_hbm, idx_v, rows_v, gsem, wsem):
        wid = lax.axis_index("s") * _NC + lax.axis_index("c")
        base = wid * r_per_w
        pltpu.sync_copy(idx_hbm.at[pl.ds(base, r_per_w), :], idx_v)

        def fire_gather(i, b):
            for j0 in range(0, S, _G):
                g = min(_G, S - j0)
                pltpu.async_copy(
                    table_hbm.at[idx_v.at[i, pl.ds(j0, g)]],
                    rows_v.at[b].at[pl.ds(j0, g)],
                    gsem.at[b],
                )

        for b in range(_NBUF):
            fire_gather(b, b)

        def step(i, carry):
            b = lax.rem(i, _NBUF)
            pltpu.make_async_copy(
                out_hbm.at[0, :, pl.ds(0, D)], rows_v.at[b], gsem.at[b]
            ).wait()
            pltpu.async_copy(
                rows_v.at[b], out_hbm.at[base + i, :, pl.ds(0, D)], wsem.at[b]
            )
            j = i - 1

            @pl.when(jnp.logical_and(j >= 0, j + _NBUF < r_per_w))
            def _():
                bj = lax.rem(j + _NBUF, _NBUF)
                pltpu.make_async_copy(
                    out_hbm.at[0, :, pl.ds(0, D)], rows_v.at[bj], wsem.at[bj]
                ).wait()
                fire_gather(j + _NBUF, bj)

            return carry

        lax.fori_loop(0, r_per_w, step, 0)

        for b in range(_NBUF):
            pltpu.make_async_copy(
                out_hbm.at[0, :, pl.ds(0, D)], rows_v.at[b], wsem.at[b]
            ).wait()

    return k(word_idxs, table)


def kernel(word_idxs, we_weight):
    v, d = we_weight.shape
    table2 = _tc_transpose_pad(we_weight.T).reshape(2 * v, d)
    out = _sc_embedding_gather(word_idxs * 2, table2)
    return out[:, :, :d]
